# baseline (speedup 1.0000x reference)
import jax
import jax.numpy as jnp
from jax import lax
from jax.experimental import pallas as pl
from jax.experimental.pallas import tpu as pltpu
from jax.experimental.pallas import tpu_sc as plsc

VOCAB = 100000
FUSION_VOCAB = 1024
D = 128
L = 16
NC, NS = 2, 16
NW = NC * NS
N_TOKENS = 4 * 8192
TPW = N_TOKENS // NW
CH = 256
N_CH = TPW // CH
NBUF = 3
GPC = CH // L
GROUPS = TPW // L
CGROUPS = GROUPS + 1


def _body(ids_hbm, emb_hbm, fus_hbm, out_hbm,
          idx_v, midx_v, rows, cpos, cfus, frow,
          sems, sem_f):
    wid = lax.axis_index("s") * NC + lax.axis_index("c")
    base = wid * TPW
    pltpu.sync_copy(ids_hbm.at[pl.ds(base, TPW)], idx_v)

    lanes = lax.iota(jnp.int32, L)

    def grp(i, cnt):
        idxv = idx_v[pl.ds(i * L, L)]
        isfus = idxv >= VOCAB
        midx_v[pl.ds(i * L, L)] = jnp.where(isfus, 0, idxv)
        posv = cnt + plsc.cumsum(jnp.where(isfus, 1, 0)) - 1
        rowp, lanep = posv >> 4, posv & (L - 1)
        plsc.store_scatter(cfus, [rowp, lanep], idxv - VOCAB, mask=isfus)
        plsc.store_scatter(cpos, [rowp, lanep], base + i * L + lanes,
                           mask=isfus)
        return cnt + plsc.all_reduce_population_count(isfus)

    gathers = [None] * N_CH
    writes = [None] * N_CH

    def fire_gather(c):
        h = pltpu.make_async_copy(
            emb_hbm.at[midx_v.at[pl.ds(c * CH, CH)]],
            rows[c % NBUF], sems[c % NBUF])
        gathers[c] = h
        pl.delay(20)
        h.start()

    def clamp_grp(i, carry):
        idxv = idx_v[pl.ds(i * L, L)]
        midx_v[pl.ds(i * L, L)] = jnp.where(idxv >= VOCAB, 0, idxv)
        return carry

    lax.fori_loop(0, GPC, clamp_grp, 0)
    fire_gather(0)

    def grp_compact(i, cnt):
        idxv = idx_v[pl.ds(i * L, L)]
        isfus = idxv >= VOCAB
        posv = cnt + plsc.cumsum(jnp.where(isfus, 1, 0)) - 1
        rowp, lanep = posv >> 4, posv & (L - 1)
        plsc.store_scatter(cfus, [rowp, lanep], idxv - VOCAB, mask=isfus)
        plsc.store_scatter(cpos, [rowp, lanep], base + i * L + lanes,
                           mask=isfus)
        return cnt + plsc.all_reduce_population_count(isfus)

    cnt = lax.fori_loop(0, GPC, grp_compact, jnp.zeros((L,), jnp.int32))
    for c in range(1, NBUF):
        cnt = lax.fori_loop(c * GPC, (c + 1) * GPC, grp, cnt)
        fire_gather(c)
    cnt = lax.fori_loop(NBUF * GPC, GROUPS, grp, cnt)

    lastj = jnp.maximum(cnt - 1, 0)
    lastpos = plsc.load_gather(cpos, [lastj >> 4, lastj & (L - 1)])
    lastfus = plsc.load_gather(cfus, [lastj >> 4, lastj & (L - 1)])
    padpos = cnt + lanes
    padmask = cnt > 0
    plsc.store_scatter(cpos, [padpos >> 4, padpos & (L - 1)], lastpos,
                       mask=padmask)
    plsc.store_scatter(cfus, [padpos >> 4, padpos & (L - 1)], lastfus,
                       mask=padmask)

    n_fus = jnp.max(cnt)
    n_fchunks = (n_fus + (L - 1)) // L

    for c in range(N_CH):
        gathers[c].wait()
        if c == 0:
            @pl.when(n_fchunks > 0)
            def _():
                pltpu.make_async_copy(
                    fus_hbm.at[cfus.at[0]], frow, sem_f).start()
        writes[c] = pltpu.make_async_copy(
            rows[c % NBUF], out_hbm.at[pl.ds(base + c * CH, CH)],
            sems[c % NBUF])
        writes[c].start()
        if 0 < c and c - 1 + NBUF < N_CH:
            writes[c - 1].wait()
            fire_gather(c - 1 + NBUF)
    for c in range(N_CH - NBUF, N_CH):
        writes[c].wait()

    @pl.when(n_fchunks > 0)
    def _():
        pltpu.make_async_copy(fus_hbm.at[cfus.at[0]], frow, sem_f).wait()
        h = pltpu.make_async_copy(frow, out_hbm.at[cpos.at[0]], sem_f)
        h.start()
        h.wait()

    def fchunk(g, carry):
        h = pltpu.make_async_copy(fus_hbm.at[cfus.at[g]], frow, sem_f)
        h.start()
        h.wait()
        h2 = pltpu.make_async_copy(frow, out_hbm.at[cpos.at[g]], sem_f)
        h2.start()
        h2.wait()
        return carry

    lax.fori_loop(1, n_fchunks, fchunk, 0)


@jax.jit
def _impl(ids, emb, fus):
    mesh = plsc.VectorSubcoreMesh(core_axis_name="c", subcore_axis_name="s",
                                  num_cores=NC, num_subcores=NS)
    return pl.kernel(
        _body,
        out_type=jax.ShapeDtypeStruct((N_TOKENS, D), jnp.float32),
        mesh=mesh,
        compiler_params=pltpu.CompilerParams(needs_layout_passes=False),
        scratch_types=[
            pltpu.VMEM((TPW,), jnp.int32),
            pltpu.VMEM((TPW,), jnp.int32),
            [pltpu.VMEM((CH, D), jnp.float32)] * NBUF,
            pltpu.VMEM((CGROUPS, L), jnp.int32),
            pltpu.VMEM((CGROUPS, L), jnp.int32),
            pltpu.VMEM((L, D), jnp.float32),
            [pltpu.SemaphoreType.DMA] * NBUF,
            pltpu.SemaphoreType.DMA,
        ],
    )(ids, emb, fus)


def kernel(input, embedding_weight, fusion_weight):
    ids = input.reshape(-1)
    out = _impl(ids, embedding_weight, fusion_weight)
    return out.reshape(input.shape + (D,))


# speedup vs baseline: 1.0428x; 1.0428x over previous
import jax
import jax.numpy as jnp
from jax import lax
from jax.experimental import pallas as pl
from jax.experimental.pallas import tpu as pltpu
from jax.experimental.pallas import tpu_sc as plsc

VOCAB = 100000
FUSION_VOCAB = 1024
D = 128
L = 16
NC, NS = 2, 16
NW = NC * NS
N_TOKENS = 4 * 8192
TPW = N_TOKENS // NW
CH = 256
N_CH = TPW // CH
NBUF = 3
GPC = CH // L
GROUPS = TPW // L
CGROUPS = GROUPS + 1


def _body(ids_hbm, emb_hbm, fus_hbm, out_hbm,
          idx_v, midx_v, rows, cpos, cfus, frow,
          sems, sem_f):
    wid = lax.axis_index("s") * NC + lax.axis_index("c")
    base = wid * TPW
    pltpu.sync_copy(ids_hbm.at[pl.ds(base, TPW)], idx_v)

    lanes = lax.iota(jnp.int32, L)

    def grp(i, cnt):
        idxv = idx_v[pl.ds(i * L, L)]
        isfus = idxv >= VOCAB
        midx_v[pl.ds(i * L, L)] = jnp.where(isfus, 0, idxv)
        posv = cnt + plsc.cumsum(jnp.where(isfus, 1, 0)) - 1
        rowp, lanep = posv >> 4, posv & (L - 1)
        plsc.store_scatter(cfus, [rowp, lanep], idxv - VOCAB, mask=isfus)
        plsc.store_scatter(cpos, [rowp, lanep], base + i * L + lanes,
                           mask=isfus)
        return cnt + plsc.all_reduce_population_count(isfus)

    gathers = [None] * N_CH
    writes = [None] * N_CH

    def fire_gather(c):
        h = pltpu.make_async_copy(
            emb_hbm.at[midx_v.at[pl.ds(c * CH, CH)]],
            rows[c % NBUF], sems[c % NBUF])
        gathers[c] = h
        h.start()

    def clamp_grp(i, carry):
        idxv = idx_v[pl.ds(i * L, L)]
        midx_v[pl.ds(i * L, L)] = jnp.where(idxv >= VOCAB, 0, idxv)
        return carry

    lax.fori_loop(0, GPC, clamp_grp, 0)

    def grp_compact(i, cnt):
        idxv = idx_v[pl.ds(i * L, L)]
        isfus = idxv >= VOCAB
        posv = cnt + plsc.cumsum(jnp.where(isfus, 1, 0)) - 1
        rowp, lanep = posv >> 4, posv & (L - 1)
        plsc.store_scatter(cfus, [rowp, lanep], idxv - VOCAB, mask=isfus)
        plsc.store_scatter(cpos, [rowp, lanep], base + i * L + lanes,
                           mask=isfus)
        return cnt + plsc.all_reduce_population_count(isfus)

    cnt = jnp.zeros((L,), jnp.int32)
    for c in range(1, N_CH):
        cnt = lax.fori_loop(c * GPC, (c + 1) * GPC, grp, cnt)
        if c - 1 < NBUF:
            fire_gather(c - 1)
    cnt = lax.fori_loop(0, GPC, grp_compact, cnt)

    lastj = jnp.maximum(cnt - 1, 0)
    lastpos = plsc.load_gather(cpos, [lastj >> 4, lastj & (L - 1)])
    lastfus = plsc.load_gather(cfus, [lastj >> 4, lastj & (L - 1)])
    padpos = cnt + lanes
    padmask = cnt > 0
    plsc.store_scatter(cpos, [padpos >> 4, padpos & (L - 1)], lastpos,
                       mask=padmask)
    plsc.store_scatter(cfus, [padpos >> 4, padpos & (L - 1)], lastfus,
                       mask=padmask)

    n_fus = jnp.max(cnt)
    n_fchunks = (n_fus + (L - 1)) // L

    for c in range(N_CH):
        gathers[c].wait()
        if c == 0:
            @pl.when(n_fchunks > 0)
            def _():
                pltpu.make_async_copy(
                    fus_hbm.at[cfus.at[0]], frow, sem_f).start()
        writes[c] = pltpu.make_async_copy(
            rows[c % NBUF], out_hbm.at[pl.ds(base + c * CH, CH)],
            sems[c % NBUF])
        writes[c].start()
        if 0 < c and c - 1 + NBUF < N_CH:
            writes[c - 1].wait()
            fire_gather(c - 1 + NBUF)
    for c in range(N_CH - NBUF, N_CH):
        writes[c].wait()

    @pl.when(n_fchunks > 0)
    def _():
        pltpu.make_async_copy(fus_hbm.at[cfus.at[0]], frow, sem_f).wait()
        h = pltpu.make_async_copy(frow, out_hbm.at[cpos.at[0]], sem_f)
        h.start()
        h.wait()

    def fchunk(g, carry):
        h = pltpu.make_async_copy(fus_hbm.at[cfus.at[g]], frow, sem_f)
        h.start()
        h.wait()
        h2 = pltpu.make_async_copy(frow, out_hbm.at[cpos.at[g]], sem_f)
        h2.start()
        h2.wait()
        return carry

    lax.fori_loop(1, n_fchunks, fchunk, 0)


@jax.jit
def _impl(ids, emb, fus):
    mesh = plsc.VectorSubcoreMesh(core_axis_name="c", subcore_axis_name="s",
                                  num_cores=NC, num_subcores=NS)
    return pl.kernel(
        _body,
        out_type=jax.ShapeDtypeStruct((N_TOKENS, D), jnp.float32),
        mesh=mesh,
        compiler_params=pltpu.CompilerParams(needs_layout_passes=False),
        scratch_types=[
            pltpu.VMEM((TPW,), jnp.int32),
            pltpu.VMEM((TPW,), jnp.int32),
            [pltpu.VMEM((CH, D), jnp.float32)] * NBUF,
            pltpu.VMEM((CGROUPS, L), jnp.int32),
            pltpu.VMEM((CGROUPS, L), jnp.int32),
            pltpu.VMEM((L, D), jnp.float32),
            [pltpu.SemaphoreType.DMA] * NBUF,
            pltpu.SemaphoreType.DMA,
        ],
    )(ids, emb, fus)


def kernel(input, embedding_weight, fusion_weight):
    ids = input.reshape(-1)
    out = _impl(ids, embedding_weight, fusion_weight)
    return out.reshape(input.shape + (D,))
